# baseline (speedup 1.0000x reference)
import functools

import jax
import jax.numpy as jnp
from jax import lax
from jax.experimental import pallas as pl
from jax.experimental.pallas import tpu as pltpu
from jax.experimental.pallas import tpu_sc as plsc

_E = 8
_K = 2
_T = 32768
_D = 1024
_TILE = 2048

_NW = 32
_CHUNK = _T // _NW
_G = _CHUNK // 16


def _tc_body(x_ref, w_ref, y_ref, lt_ref):
    x = x_ref[...]
    w = w_ref[...]
    logits = jax.lax.dot_general(
        x, w, (((1,), (0,)), ((), ())), preferred_element_type=jnp.float32
    )
    lt_ref[...] = jax.lax.dot_general(
        w, x, (((0,), (1,)), ((), ())), preferred_element_type=jnp.float32
    )

    iota_e = jax.lax.broadcasted_iota(jnp.int32, logits.shape, 1)
    v0 = jnp.max(logits, axis=-1, keepdims=True)
    first = jnp.min(jnp.where(logits == v0, iota_e, _E), axis=-1, keepdims=True)
    masked = jnp.where(iota_e == first, -jnp.inf, logits)
    v1 = jnp.max(masked, axis=-1, keepdims=True)

    e1 = jnp.exp(v1 - v0)
    s = 1.0 + e1
    wsum = 1.0 / s + e1 / s
    y_ref[...] = x * wsum


def _sc_body(lt_hbm, out_hbm, lvmem, accv, sem):
    wid = lax.axis_index("s") * 2 + lax.axis_index("c")
    base = wid * _CHUNK
    pltpu.sync_copy(lt_hbm.at[:, pl.ds(base, _CHUNK)], lvmem)
    one = jnp.ones((16,), jnp.int32)
    zero = jnp.zeros((16,), jnp.int32)
    for e in range(_E):
        accv[e, :] = zero

    def group(g, c):
        vs = [lvmem[e, pl.ds(g * 16, 16)] for e in range(_E)]
        for e in range(_E):
            rank = zero
            for j in range(_E):
                if j == e:
                    continue
                rank = rank + jnp.where(vs[j] > vs[e], one, zero)
                if j < e:
                    rank = rank + jnp.where(vs[j] == vs[e], one, zero)
            plsc.addupdate(accv.at[e, :], jnp.where(rank < _K, one, zero))
        return c

    lax.fori_loop(0, _G, group, 0)
    pltpu.sync_copy(accv, out_hbm.at[wid])


def _sc_hist(logits_t):
    mesh = plsc.VectorSubcoreMesh(core_axis_name="c", subcore_axis_name="s")
    f = functools.partial(
        pl.kernel,
        out_type=jax.ShapeDtypeStruct((_NW, _E, 16), jnp.int32),
        mesh=mesh,
        scratch_types=[
            pltpu.VMEM((_E, _CHUNK), jnp.float32),
            pltpu.VMEM((_E, 16), jnp.int32),
            pltpu.SemaphoreType.DMA,
        ],
    )(_sc_body)
    return f(logits_t)


def kernel(x, router_weight):
    grid = (_T // _TILE,)
    combined, logits_t = pl.pallas_call(
        _tc_body,
        grid=grid,
        in_specs=[
            pl.BlockSpec((_TILE, _D), lambda i: (i, 0)),
            pl.BlockSpec((_D, _E), lambda i: (0, 0)),
        ],
        out_specs=[
            pl.BlockSpec((_TILE, _D), lambda i: (i, 0)),
            pl.BlockSpec((_E, _TILE), lambda i: (0, i)),
        ],
        out_shape=[
            jax.ShapeDtypeStruct((_T, _D), jnp.float32),
            jax.ShapeDtypeStruct((_E, _T), jnp.float32),
        ],
        compiler_params=pltpu.CompilerParams(
            dimension_semantics=("parallel",),
        ),
    )(x, router_weight)
    part = _sc_hist(logits_t)
    return combined, jnp.sum(part, axis=(0, 2))


# speedup vs baseline: 1.2781x; 1.2781x over previous
import jax
import jax.numpy as jnp
from jax.experimental import pallas as pl
from jax.experimental.pallas import tpu as pltpu

_E = 8
_K = 2
_T = 32768
_D = 1024
_TILE = 2048


def _fused_body(x_ref, w_ref, y_ref, hist_ref):
    x = x_ref[...]
    w = w_ref[...]
    logits = jax.lax.dot_general(
        x, w, (((1,), (0,)), ((), ())), preferred_element_type=jnp.float32
    )

    v0 = jnp.max(logits, axis=-1, keepdims=True)
    masked = jnp.where(logits == v0, -jnp.inf, logits)
    v1 = jnp.max(masked, axis=-1, keepdims=True)

    e1 = jnp.exp(v1 - v0)
    s = 1.0 + e1
    wsum = 1.0 / s + e1 / s
    y_ref[...] = x * wsum

    in2 = jnp.where(logits >= v1, jnp.int32(1), jnp.int32(0))
    hist_ref[0] = jnp.sum(in2, axis=0, keepdims=True)


def kernel(x, router_weight):
    grid = (_T // _TILE,)
    combined, hist = pl.pallas_call(
        _fused_body,
        grid=grid,
        in_specs=[
            pl.BlockSpec((_TILE, _D), lambda i: (i, 0)),
            pl.BlockSpec((_D, _E), lambda i: (0, 0)),
        ],
        out_specs=[
            pl.BlockSpec((_TILE, _D), lambda i: (i, 0)),
            pl.BlockSpec((1, 1, _E), lambda i: (i, 0, 0)),
        ],
        out_shape=[
            jax.ShapeDtypeStruct((_T, _D), jnp.float32),
            jax.ShapeDtypeStruct((grid[0], 1, _E), jnp.int32),
        ],
        compiler_params=pltpu.CompilerParams(
            dimension_semantics=("parallel",),
        ),
    )(x, router_weight)
    return combined, jnp.sum(hist[:, 0, :], axis=0)
